# baseline (speedup 1.0000x reference)
import functools

import jax
import jax.numpy as jnp
from jax import lax
from jax.experimental import pallas as pl
from jax.experimental.pallas import tpu as pltpu
from jax.experimental.pallas import tpu_sc as plsc

N = 10000
E = 320000
D = 128
N_PAD = 10240
NC = 2
NS = 16
NW = NC * NS

EPW = E // NW
B = 112
NFULL = EPW // B
TAIL = EPW - NFULL * B
RPT_M = N // NS

EPT = E // NS
BD = 128
CPT = EPT // BD
BD_TAIL = EPT - CPT * BD
RPT_D = N_PAD // NS

RBLK = 1024


def _sc_mesh():
    return plsc.VectorSubcoreMesh(
        core_axis_name="c", subcore_axis_name="s", num_cores=NC, num_subcores=NS
    )


def _deg_kernel_body(src_hbm, dst_hbm, ones_hbm, zeros_hbm, deg_hbm,
                     idx_v, ones_v, dacc, semA, semB):
    c = lax.axis_index("c")
    s = lax.axis_index("s")
    pltpu.sync_copy(
        zeros_hbm.at[pl.ds(s * RPT_D, RPT_D)], dacc.at[pl.ds(s * RPT_D, RPT_D)]
    )
    pltpu.sync_copy(ones_hbm, ones_v)

    @pl.when(c == 0)
    def _():
        pltpu.sync_copy(src_hbm.at[pl.ds(s * EPT, EPT)], idx_v)

    @pl.when(c == 1)
    def _():
        pltpu.sync_copy(dst_hbm.at[pl.ds(s * EPT, EPT)], idx_v)

    plsc.subcore_barrier()

    def _scat(j, sem):
        return pltpu.async_copy(
            ones_v, dacc.at[idx_v.at[pl.ds(j * BD, BD)]], sem, add=True
        )

    def _scat_wait(j, sem):
        pltpu.make_async_copy(
            ones_v, dacc.at[idx_v.at[pl.ds(j * BD, BD)]], sem
        ).wait()

    _scat(0, semA)

    def body(g, carry):
        j = 2 * g
        _scat(j + 1, semB)
        _scat_wait(j, semA)
        _scat(j + 2, semA)
        _scat_wait(j + 1, semB)
        return carry

    lax.fori_loop(0, CPT // 2 - 1, body, 0)
    _scat_wait(CPT - 2, semA)
    _scat(CPT - 1, semB)
    pltpu.async_copy(
        ones_v.at[pl.ds(0, BD_TAIL)],
        dacc.at[idx_v.at[pl.ds(CPT * BD, BD_TAIL)]],
        semA,
        add=True,
    )
    _scat_wait(CPT - 1, semB)
    pltpu.make_async_copy(
        ones_v.at[pl.ds(0, BD_TAIL)],
        dacc.at[idx_v.at[pl.ds(CPT * BD, BD_TAIL)]],
        semA,
    ).wait()

    plsc.subcore_barrier()
    pltpu.sync_copy(
        dacc.at[pl.ds(s * RPT_D, RPT_D)], deg_hbm.at[c, pl.ds(s * RPT_D, RPT_D)]
    )


def _make_deg_kernel():
    return functools.partial(
        pl.kernel,
        out_type=jax.ShapeDtypeStruct((NC, N_PAD, 8), jnp.float32),
        mesh=_sc_mesh(),
        scratch_types=[
            pltpu.VMEM((EPT,), jnp.int32),
            pltpu.VMEM((BD, 8), jnp.float32),
            pltpu.VMEM_SHARED((N_PAD, 8), jnp.float32),
            pltpu.SemaphoreType.DMA,
            pltpu.SemaphoreType.DMA,
        ],
        compiler_params=pltpu.CompilerParams(use_tc_tiling_on_sc=False),
    )(_deg_kernel_body)


def _msg_kernel_body(
    h_hbm, src_hbm, dst_hbm, zeros_hbm, out_hbm,
    src_v, dst_v, buf0, buf1, acc, sem0, sem1, sems0, sems1
):
    c = lax.axis_index("c")
    s = lax.axis_index("s")
    w = s * NC + c
    pltpu.sync_copy(src_hbm.at[pl.ds(w * EPW, EPW)], src_v)
    pltpu.sync_copy(dst_hbm.at[pl.ds(w * EPW, EPW)], dst_v)
    pltpu.sync_copy(
        zeros_hbm.at[pl.ds(s * RPT_M, RPT_M)], acc.at[pl.ds(s * RPT_M, RPT_M)]
    )
    plsc.subcore_barrier()

    def _gat(j, buf, sem):
        pltpu.async_copy(h_hbm.at[src_v.at[pl.ds(j * B, B)]], buf, sem)

    def _gat_wait(j, buf, sem):
        pltpu.make_async_copy(
            h_hbm.at[src_v.at[pl.ds(j * B, B)]], buf, sem
        ).wait()

    def _scat(j, buf, sem):
        pltpu.async_copy(buf, acc.at[dst_v.at[pl.ds(j * B, B)]], sem, add=True)

    def _scat_wait(j, buf, sem):
        pltpu.make_async_copy(
            buf, acc.at[dst_v.at[pl.ds(j * B, B)]], sem
        ).wait()

    _gat(0, buf0, sem0)
    _gat(1, buf1, sem1)

    def body(g, carry):
        j = 2 * g
        _gat_wait(j, buf0, sem0)
        _scat(j, buf0, sems0)
        _gat_wait(j + 1, buf1, sem1)
        _scat(j + 1, buf1, sems1)
        _scat_wait(j, buf0, sems0)
        _gat(j + 2, buf0, sem0)
        _scat_wait(j + 1, buf1, sems1)
        _gat(j + 3, buf1, sem1)
        return carry

    lax.fori_loop(0, (NFULL - 3) // 2, body, 0)
    j = NFULL - 3
    _gat_wait(j, buf0, sem0)
    _scat(j, buf0, sems0)
    _gat_wait(j + 1, buf1, sem1)
    _scat(j + 1, buf1, sems1)
    _scat_wait(j, buf0, sems0)
    _gat(j + 2, buf0, sem0)
    _scat_wait(j + 1, buf1, sems1)
    pltpu.async_copy(
        h_hbm.at[src_v.at[pl.ds(NFULL * B, TAIL)]], buf1.at[pl.ds(0, TAIL)], sem1
    )
    _gat_wait(j + 2, buf0, sem0)
    _scat(j + 2, buf0, sems0)
    pltpu.make_async_copy(
        h_hbm.at[src_v.at[pl.ds(NFULL * B, TAIL)]], buf1.at[pl.ds(0, TAIL)], sem1
    ).wait()
    pltpu.sync_copy(
        buf1.at[pl.ds(0, TAIL)],
        acc.at[dst_v.at[pl.ds(NFULL * B, TAIL)]],
        add=True,
    )
    _scat_wait(j + 2, buf0, sems0)
    plsc.subcore_barrier()
    pltpu.sync_copy(
        acc.at[pl.ds(s * RPT_M, RPT_M)], out_hbm.at[c, pl.ds(s * RPT_M, RPT_M)]
    )


def _make_msg_kernel():
    return functools.partial(
        pl.kernel,
        out_type=jax.ShapeDtypeStruct((NC, N, D), jnp.float32),
        mesh=_sc_mesh(),
        scratch_types=[
            pltpu.VMEM((EPW,), jnp.int32),
            pltpu.VMEM((EPW,), jnp.int32),
            pltpu.VMEM((B, D), jnp.float32),
            pltpu.VMEM((B, D), jnp.float32),
            pltpu.VMEM_SHARED((N, D), jnp.float32),
            pltpu.SemaphoreType.DMA,
            pltpu.SemaphoreType.DMA,
            pltpu.SemaphoreType.DMA,
            pltpu.SemaphoreType.DMA,
        ],
        compiler_params=pltpu.CompilerParams(use_tc_tiling_on_sc=False),
    )(_msg_kernel_body)


def _norms(deg_blk):
    nrm = lax.rsqrt(jnp.maximum(deg_blk[:, :, 0:1], 1.0))
    return nrm[0], nrm[1]


def _tc1_body(x_ref, deg_ref, w_ref, o_ref):
    ns, _ = _norms(deg_ref[...])
    o_ref[...] = jnp.dot(
        x_ref[...] * ns, w_ref[...], preferred_element_type=jnp.float32
    )


def _tc_mid_body(p_ref, deg_ref, b_ref, w_ref, o_ref):
    ns, nd = _norms(deg_ref[...])
    t = (p_ref[0] + p_ref[1]) * nd + b_ref[...]
    t = jnp.where(t > 0.0, t, jnp.exp(jnp.minimum(t, 0.0)) - 1.0)
    o_ref[...] = jnp.dot(t * ns, w_ref[...], preferred_element_type=jnp.float32)


def _tc_out_body(p_ref, deg_ref, b_ref, o_ref):
    _, nd = _norms(deg_ref[...])
    o_ref[...] = (p_ref[0] + p_ref[1]) * nd + b_ref[...]


_GRID = (N_PAD // RBLK,)
_SPEC_ROWS = pl.BlockSpec((RBLK, D), lambda i: (i, 0))
_SPEC_DEG = pl.BlockSpec((NC, RBLK, 8), lambda i: (0, i, 0))
_SPEC_P = pl.BlockSpec((NC, RBLK, D), lambda i: (0, i, 0))
_SPEC_W = pl.BlockSpec((D, D), lambda i: (0, 0))
_SPEC_B = pl.BlockSpec((1, D), lambda i: (0, 0))
_OUT_ROWS = jax.ShapeDtypeStruct((N_PAD, D), jnp.float32)


def kernel(x, edge_index, W1, b1, W2, b2):
    src = edge_index[0]
    dst = edge_index[1]
    zeros_nd = jnp.zeros((N, D), jnp.float32)
    zeros_deg = jnp.zeros((N_PAD, 8), jnp.float32)
    ones_bd = jnp.ones((BD, 8), jnp.float32)

    deg3 = _make_deg_kernel()(src, dst, ones_bd, zeros_deg)

    h1 = pl.pallas_call(
        _tc1_body,
        grid=_GRID,
        in_specs=[_SPEC_ROWS, _SPEC_DEG, _SPEC_W],
        out_specs=_SPEC_ROWS,
        out_shape=_OUT_ROWS,
    )(x, deg3, W1)

    msg = _make_msg_kernel()
    p1 = msg(h1, src, dst, zeros_nd)

    h2 = pl.pallas_call(
        _tc_mid_body,
        grid=_GRID,
        in_specs=[_SPEC_P, _SPEC_DEG, _SPEC_B, _SPEC_W],
        out_specs=_SPEC_ROWS,
        out_shape=_OUT_ROWS,
    )(p1, deg3, b1.reshape(1, D), W2)

    p2 = msg(h2, src, dst, zeros_nd)

    logits = pl.pallas_call(
        _tc_out_body,
        grid=_GRID,
        in_specs=[_SPEC_P, _SPEC_DEG, _SPEC_B],
        out_specs=_SPEC_ROWS,
        out_shape=jax.ShapeDtypeStruct((N, D), jnp.float32),
    )(p2, deg3, b2.reshape(1, D))

    return logits


# speedup vs baseline: 1.2395x; 1.2395x over previous
import functools

import jax
import jax.numpy as jnp
from jax import lax
from jax.experimental import pallas as pl
from jax.experimental.pallas import tpu as pltpu
from jax.experimental.pallas import tpu_sc as plsc

N = 10000
E = 320000
D = 128
N_PAD = 10240
NC = 2
NS = 16
NW = NC * NS

EPW = E // NW
B = 112
NFULL = EPW // B
TAIL = EPW - NFULL * B
RPT_M = N // NS

EPT = E // NS
BD = 128
CPT = EPT // BD
BD_TAIL = EPT - CPT * BD
RPT_D = N_PAD // NS

RBLK = 1024


def _sc_mesh():
    return plsc.VectorSubcoreMesh(
        core_axis_name="c", subcore_axis_name="s", num_cores=NC, num_subcores=NS
    )


def _deg_kernel_body(src_hbm, dst_hbm, ones_hbm, zeros_hbm, deg_hbm,
                     idx_v, ones_v, dacc, semA, semB):
    c = lax.axis_index("c")
    s = lax.axis_index("s")
    pltpu.sync_copy(
        zeros_hbm.at[pl.ds(s * RPT_D, RPT_D)], dacc.at[pl.ds(s * RPT_D, RPT_D)]
    )
    pltpu.sync_copy(ones_hbm, ones_v)

    @pl.when(c == 0)
    def _():
        pltpu.sync_copy(src_hbm.at[pl.ds(s * EPT, EPT)], idx_v)

    @pl.when(c == 1)
    def _():
        pltpu.sync_copy(dst_hbm.at[pl.ds(s * EPT, EPT)], idx_v)

    plsc.subcore_barrier()

    def _scat(j, sem):
        return pltpu.async_copy(
            ones_v, dacc.at[idx_v.at[pl.ds(j * BD, BD)]], sem, add=True
        )

    def _scat_wait(j, sem):
        pltpu.make_async_copy(
            ones_v, dacc.at[idx_v.at[pl.ds(j * BD, BD)]], sem
        ).wait()

    _scat(0, semA)

    def body(g, carry):
        j = 2 * g
        _scat(j + 1, semB)
        _scat_wait(j, semA)
        _scat(j + 2, semA)
        _scat_wait(j + 1, semB)
        return carry

    lax.fori_loop(0, CPT // 2 - 1, body, 0)
    _scat_wait(CPT - 2, semA)
    _scat(CPT - 1, semB)
    pltpu.async_copy(
        ones_v.at[pl.ds(0, BD_TAIL)],
        dacc.at[idx_v.at[pl.ds(CPT * BD, BD_TAIL)]],
        semA,
        add=True,
    )
    _scat_wait(CPT - 1, semB)
    pltpu.make_async_copy(
        ones_v.at[pl.ds(0, BD_TAIL)],
        dacc.at[idx_v.at[pl.ds(CPT * BD, BD_TAIL)]],
        semA,
    ).wait()

    plsc.subcore_barrier()
    pltpu.sync_copy(
        dacc.at[pl.ds(s * RPT_D, RPT_D)], deg_hbm.at[c, pl.ds(s * RPT_D, RPT_D)]
    )


def _make_deg_kernel():
    return functools.partial(
        pl.kernel,
        out_type=jax.ShapeDtypeStruct((NC, N_PAD, 8), jnp.float32),
        mesh=_sc_mesh(),
        scratch_types=[
            pltpu.VMEM((EPT,), jnp.int32),
            pltpu.VMEM((BD, 8), jnp.float32),
            pltpu.VMEM_SHARED((N_PAD, 8), jnp.float32),
            pltpu.SemaphoreType.DMA,
            pltpu.SemaphoreType.DMA,
        ],
        compiler_params=pltpu.CompilerParams(use_tc_tiling_on_sc=False),
    )(_deg_kernel_body)


def _msg_kernel_body(
    h_hbm, src_hbm, dst_hbm, zeros_hbm, out_hbm,
    src_v, dst_v, buf0, buf1, acc, sem0, sem1
):
    c = lax.axis_index("c")
    s = lax.axis_index("s")
    w = s * NC + c
    pltpu.sync_copy(src_hbm.at[pl.ds(w * EPW, EPW)], src_v)
    pltpu.sync_copy(dst_hbm.at[pl.ds(w * EPW, EPW)], dst_v)
    pltpu.sync_copy(
        zeros_hbm.at[pl.ds(s * RPT_M, RPT_M)], acc.at[pl.ds(s * RPT_M, RPT_M)]
    )
    plsc.subcore_barrier()

    pltpu.async_copy(h_hbm.at[src_v.at[pl.ds(0, B)]], buf0, sem0)

    def body(g, carry):
        j = 2 * g
        pltpu.async_copy(h_hbm.at[src_v.at[pl.ds((j + 1) * B, B)]], buf1, sem1)
        pltpu.make_async_copy(
            h_hbm.at[src_v.at[pl.ds(j * B, B)]], buf0, sem0
        ).wait()
        pltpu.sync_copy(buf0, acc.at[dst_v.at[pl.ds(j * B, B)]], add=True)
        pltpu.async_copy(h_hbm.at[src_v.at[pl.ds((j + 2) * B, B)]], buf0, sem0)
        pltpu.make_async_copy(
            h_hbm.at[src_v.at[pl.ds((j + 1) * B, B)]], buf1, sem1
        ).wait()
        pltpu.sync_copy(buf1, acc.at[dst_v.at[pl.ds((j + 1) * B, B)]], add=True)
        return carry

    lax.fori_loop(0, (NFULL - 1) // 2, body, 0)
    pltpu.async_copy(
        h_hbm.at[src_v.at[pl.ds(NFULL * B, TAIL)]], buf1.at[pl.ds(0, TAIL)], sem1
    )
    pltpu.make_async_copy(
        h_hbm.at[src_v.at[pl.ds((NFULL - 1) * B, B)]], buf0, sem0
    ).wait()
    pltpu.sync_copy(buf0, acc.at[dst_v.at[pl.ds((NFULL - 1) * B, B)]], add=True)
    pltpu.make_async_copy(
        h_hbm.at[src_v.at[pl.ds(NFULL * B, TAIL)]], buf1.at[pl.ds(0, TAIL)], sem1
    ).wait()
    pltpu.sync_copy(
        buf1.at[pl.ds(0, TAIL)],
        acc.at[dst_v.at[pl.ds(NFULL * B, TAIL)]],
        add=True,
    )
    plsc.subcore_barrier()
    pltpu.sync_copy(
        acc.at[pl.ds(s * RPT_M, RPT_M)], out_hbm.at[c, pl.ds(s * RPT_M, RPT_M)]
    )


def _make_msg_kernel():
    return functools.partial(
        pl.kernel,
        out_type=jax.ShapeDtypeStruct((NC, N, D), jnp.float32),
        mesh=_sc_mesh(),
        scratch_types=[
            pltpu.VMEM((EPW,), jnp.int32),
            pltpu.VMEM((EPW,), jnp.int32),
            pltpu.VMEM((B, D), jnp.float32),
            pltpu.VMEM((B, D), jnp.float32),
            pltpu.VMEM_SHARED((N, D), jnp.float32),
            pltpu.SemaphoreType.DMA,
            pltpu.SemaphoreType.DMA,
        ],
        compiler_params=pltpu.CompilerParams(use_tc_tiling_on_sc=False),
    )(_msg_kernel_body)


def _norms(deg_blk):
    nrm = lax.rsqrt(jnp.maximum(deg_blk[:, :, 0:1], 1.0))
    return nrm[0], nrm[1]


def _tc1_body(x_ref, deg_ref, w_ref, o_ref):
    ns, _ = _norms(deg_ref[...])
    o_ref[...] = jnp.dot(
        x_ref[...] * ns, w_ref[...], preferred_element_type=jnp.float32
    )


def _tc_mid_body(p_ref, deg_ref, b_ref, w_ref, o_ref):
    ns, nd = _norms(deg_ref[...])
    t = (p_ref[0] + p_ref[1]) * nd + b_ref[...]
    t = jnp.where(t > 0.0, t, jnp.exp(jnp.minimum(t, 0.0)) - 1.0)
    o_ref[...] = jnp.dot(t * ns, w_ref[...], preferred_element_type=jnp.float32)


def _tc_out_body(p_ref, deg_ref, b_ref, o_ref):
    _, nd = _norms(deg_ref[...])
    o_ref[...] = (p_ref[0] + p_ref[1]) * nd + b_ref[...]


_GRID = (N_PAD // RBLK,)
_SPEC_ROWS = pl.BlockSpec((RBLK, D), lambda i: (i, 0))
_SPEC_DEG = pl.BlockSpec((NC, RBLK, 8), lambda i: (0, i, 0))
_SPEC_P = pl.BlockSpec((NC, RBLK, D), lambda i: (0, i, 0))
_SPEC_W = pl.BlockSpec((D, D), lambda i: (0, 0))
_SPEC_B = pl.BlockSpec((1, D), lambda i: (0, 0))
_OUT_ROWS = jax.ShapeDtypeStruct((N_PAD, D), jnp.float32)


def kernel(x, edge_index, W1, b1, W2, b2):
    src = edge_index[0]
    dst = edge_index[1]
    zeros_nd = jnp.zeros((N, D), jnp.float32)
    zeros_deg = jnp.zeros((N_PAD, 8), jnp.float32)
    ones_bd = jnp.ones((BD, 8), jnp.float32)

    deg3 = _make_deg_kernel()(src, dst, ones_bd, zeros_deg)

    h1 = pl.pallas_call(
        _tc1_body,
        grid=_GRID,
        in_specs=[_SPEC_ROWS, _SPEC_DEG, _SPEC_W],
        out_specs=_SPEC_ROWS,
        out_shape=_OUT_ROWS,
    )(x, deg3, W1)

    msg = _make_msg_kernel()
    p1 = msg(h1, src, dst, zeros_nd)

    h2 = pl.pallas_call(
        _tc_mid_body,
        grid=_GRID,
        in_specs=[_SPEC_P, _SPEC_DEG, _SPEC_B, _SPEC_W],
        out_specs=_SPEC_ROWS,
        out_shape=_OUT_ROWS,
    )(p1, deg3, b1.reshape(1, D), W2)

    p2 = msg(h2, src, dst, zeros_nd)

    logits = pl.pallas_call(
        _tc_out_body,
        grid=_GRID,
        in_specs=[_SPEC_P, _SPEC_DEG, _SPEC_B],
        out_specs=_SPEC_ROWS,
        out_shape=jax.ShapeDtypeStruct((N, D), jnp.float32),
    )(p2, deg3, b2.reshape(1, D))

    return logits


# speedup vs baseline: 1.2694x; 1.0242x over previous
import functools

import jax
import jax.numpy as jnp
from jax import lax
from jax.experimental import pallas as pl
from jax.experimental.pallas import tpu as pltpu
from jax.experimental.pallas import tpu_sc as plsc

N = 10000
E = 320000
D = 128
N_PAD = 10240
NC = 2
NS = 16
NW = NC * NS

EPW = E // NW
B = 112
NFULL = EPW // B
TAIL = EPW - NFULL * B
RPT_M = N // NS

EPT = E // NS
BD = 128
CPT = EPT // BD
BD_TAIL = EPT - CPT * BD
RPT_D = N_PAD // NS

RBLK = 2048


def _sc_mesh():
    return plsc.VectorSubcoreMesh(
        core_axis_name="c", subcore_axis_name="s", num_cores=NC, num_subcores=NS
    )


def _deg_kernel_body(src_hbm, dst_hbm, ones_hbm, zeros_hbm, deg_hbm,
                     idx_v, ones_v, dacc, semA, semB):
    c = lax.axis_index("c")
    s = lax.axis_index("s")
    pltpu.sync_copy(
        zeros_hbm.at[pl.ds(s * RPT_D, RPT_D)], dacc.at[pl.ds(s * RPT_D, RPT_D)]
    )
    pltpu.sync_copy(ones_hbm, ones_v)

    @pl.when(c == 0)
    def _():
        pltpu.sync_copy(src_hbm.at[pl.ds(s * EPT, EPT)], idx_v)

    @pl.when(c == 1)
    def _():
        pltpu.sync_copy(dst_hbm.at[pl.ds(s * EPT, EPT)], idx_v)

    plsc.subcore_barrier()

    def _scat(j, sem):
        return pltpu.async_copy(
            ones_v, dacc.at[idx_v.at[pl.ds(j * BD, BD)]], sem, add=True
        )

    def _scat_wait(j, sem):
        pltpu.make_async_copy(
            ones_v, dacc.at[idx_v.at[pl.ds(j * BD, BD)]], sem
        ).wait()

    _scat(0, semA)

    def body(g, carry):
        j = 2 * g
        _scat(j + 1, semB)
        _scat_wait(j, semA)
        _scat(j + 2, semA)
        _scat_wait(j + 1, semB)
        return carry

    lax.fori_loop(0, CPT // 2 - 1, body, 0)
    _scat_wait(CPT - 2, semA)
    _scat(CPT - 1, semB)
    pltpu.async_copy(
        ones_v.at[pl.ds(0, BD_TAIL)],
        dacc.at[idx_v.at[pl.ds(CPT * BD, BD_TAIL)]],
        semA,
        add=True,
    )
    _scat_wait(CPT - 1, semB)
    pltpu.make_async_copy(
        ones_v.at[pl.ds(0, BD_TAIL)],
        dacc.at[idx_v.at[pl.ds(CPT * BD, BD_TAIL)]],
        semA,
    ).wait()

    plsc.subcore_barrier()
    pltpu.sync_copy(
        dacc.at[pl.ds(s * RPT_D, RPT_D)], deg_hbm.at[c, pl.ds(s * RPT_D, RPT_D)]
    )


def _make_deg_kernel():
    return functools.partial(
        pl.kernel,
        out_type=jax.ShapeDtypeStruct((NC, N_PAD, 8), jnp.float32),
        mesh=_sc_mesh(),
        scratch_types=[
            pltpu.VMEM((EPT,), jnp.int32),
            pltpu.VMEM((BD, 8), jnp.float32),
            pltpu.VMEM_SHARED((N_PAD, 8), jnp.float32),
            pltpu.SemaphoreType.DMA,
            pltpu.SemaphoreType.DMA,
        ],
        compiler_params=pltpu.CompilerParams(use_tc_tiling_on_sc=False),
    )(_deg_kernel_body)


def _msg_kernel_body(
    h_hbm, src_hbm, dst_hbm, zeros_hbm, out_hbm,
    src_v, dst_v, buf0, buf1, acc, sem0, sem1
):
    c = lax.axis_index("c")
    s = lax.axis_index("s")
    w = s * NC + c
    pltpu.sync_copy(src_hbm.at[pl.ds(w * EPW, EPW)], src_v)
    pltpu.sync_copy(dst_hbm.at[pl.ds(w * EPW, EPW)], dst_v)
    pltpu.sync_copy(
        zeros_hbm.at[pl.ds(s * RPT_M, RPT_M)], acc.at[pl.ds(s * RPT_M, RPT_M)]
    )
    plsc.subcore_barrier()

    pltpu.async_copy(h_hbm.at[src_v.at[pl.ds(0, B)]], buf0, sem0)

    def body(g, carry):
        j = 2 * g
        pltpu.async_copy(h_hbm.at[src_v.at[pl.ds((j + 1) * B, B)]], buf1, sem1)
        pltpu.make_async_copy(
            h_hbm.at[src_v.at[pl.ds(j * B, B)]], buf0, sem0
        ).wait()
        pltpu.sync_copy(buf0, acc.at[dst_v.at[pl.ds(j * B, B)]], add=True)
        pltpu.async_copy(h_hbm.at[src_v.at[pl.ds((j + 2) * B, B)]], buf0, sem0)
        pltpu.make_async_copy(
            h_hbm.at[src_v.at[pl.ds((j + 1) * B, B)]], buf1, sem1
        ).wait()
        pltpu.sync_copy(buf1, acc.at[dst_v.at[pl.ds((j + 1) * B, B)]], add=True)
        return carry

    lax.fori_loop(0, (NFULL - 1) // 2, body, 0)
    pltpu.async_copy(
        h_hbm.at[src_v.at[pl.ds(NFULL * B, TAIL)]], buf1.at[pl.ds(0, TAIL)], sem1
    )
    pltpu.make_async_copy(
        h_hbm.at[src_v.at[pl.ds((NFULL - 1) * B, B)]], buf0, sem0
    ).wait()
    pltpu.sync_copy(buf0, acc.at[dst_v.at[pl.ds((NFULL - 1) * B, B)]], add=True)
    pltpu.make_async_copy(
        h_hbm.at[src_v.at[pl.ds(NFULL * B, TAIL)]], buf1.at[pl.ds(0, TAIL)], sem1
    ).wait()
    pltpu.sync_copy(
        buf1.at[pl.ds(0, TAIL)],
        acc.at[dst_v.at[pl.ds(NFULL * B, TAIL)]],
        add=True,
    )
    plsc.subcore_barrier()
    pltpu.sync_copy(
        acc.at[pl.ds(s * RPT_M, RPT_M)], out_hbm.at[c, pl.ds(s * RPT_M, RPT_M)]
    )


def _make_msg_kernel():
    return functools.partial(
        pl.kernel,
        out_type=jax.ShapeDtypeStruct((NC, N, D), jnp.float32),
        mesh=_sc_mesh(),
        scratch_types=[
            pltpu.VMEM((EPW,), jnp.int32),
            pltpu.VMEM((EPW,), jnp.int32),
            pltpu.VMEM((B, D), jnp.float32),
            pltpu.VMEM((B, D), jnp.float32),
            pltpu.VMEM_SHARED((N, D), jnp.float32),
            pltpu.SemaphoreType.DMA,
            pltpu.SemaphoreType.DMA,
        ],
        compiler_params=pltpu.CompilerParams(use_tc_tiling_on_sc=False),
    )(_msg_kernel_body)


def _norms(deg_blk):
    nrm = lax.rsqrt(jnp.maximum(deg_blk[:, :, 0:1], 1.0))
    return nrm[0], nrm[1]


def _tc1_body(x_ref, deg_ref, w_ref, o_ref):
    ns, _ = _norms(deg_ref[...])
    o_ref[...] = jnp.dot(
        x_ref[...] * ns, w_ref[...], preferred_element_type=jnp.float32
    )


def _tc_mid_body(p_ref, deg_ref, b_ref, w_ref, o_ref):
    ns, nd = _norms(deg_ref[...])
    t = (p_ref[0] + p_ref[1]) * nd + b_ref[...]
    t = jnp.where(t > 0.0, t, jnp.exp(jnp.minimum(t, 0.0)) - 1.0)
    o_ref[...] = jnp.dot(t * ns, w_ref[...], preferred_element_type=jnp.float32)


def _tc_out_body(p_ref, deg_ref, b_ref, o_ref):
    _, nd = _norms(deg_ref[...])
    o_ref[...] = (p_ref[0] + p_ref[1]) * nd + b_ref[...]


_GRID = (N_PAD // RBLK,)
_SPEC_ROWS = pl.BlockSpec((RBLK, D), lambda i: (i, 0))
_SPEC_DEG = pl.BlockSpec((NC, RBLK, 8), lambda i: (0, i, 0))
_SPEC_P = pl.BlockSpec((NC, RBLK, D), lambda i: (0, i, 0))
_SPEC_W = pl.BlockSpec((D, D), lambda i: (0, 0))
_SPEC_B = pl.BlockSpec((1, D), lambda i: (0, 0))
_OUT_ROWS = jax.ShapeDtypeStruct((N_PAD, D), jnp.float32)


def kernel(x, edge_index, W1, b1, W2, b2):
    src = edge_index[0]
    dst = edge_index[1]
    zeros_nd = jnp.zeros((N, D), jnp.float32)
    zeros_deg = jnp.zeros((N_PAD, 8), jnp.float32)
    ones_bd = jnp.ones((BD, 8), jnp.float32)

    deg3 = _make_deg_kernel()(src, dst, ones_bd, zeros_deg)

    h1 = pl.pallas_call(
        _tc1_body,
        grid=_GRID,
        in_specs=[_SPEC_ROWS, _SPEC_DEG, _SPEC_W],
        out_specs=_SPEC_ROWS,
        out_shape=_OUT_ROWS,
    )(x, deg3, W1)

    msg = _make_msg_kernel()
    p1 = msg(h1, src, dst, zeros_nd)

    h2 = pl.pallas_call(
        _tc_mid_body,
        grid=_GRID,
        in_specs=[_SPEC_P, _SPEC_DEG, _SPEC_B, _SPEC_W],
        out_specs=_SPEC_ROWS,
        out_shape=_OUT_ROWS,
    )(p1, deg3, b1.reshape(1, D), W2)

    p2 = msg(h2, src, dst, zeros_nd)

    logits = pl.pallas_call(
        _tc_out_body,
        grid=_GRID,
        in_specs=[_SPEC_P, _SPEC_DEG, _SPEC_B],
        out_specs=_SPEC_ROWS,
        out_shape=jax.ShapeDtypeStruct((N, D), jnp.float32),
    )(p2, deg3, b2.reshape(1, D))

    return logits


# speedup vs baseline: 1.2741x; 1.0037x over previous
import functools

import jax
import jax.numpy as jnp
from jax import lax
from jax.experimental import pallas as pl
from jax.experimental.pallas import tpu as pltpu
from jax.experimental.pallas import tpu_sc as plsc

N = 10000
E = 320000
D = 128
N_PAD = 10240
NC = 2
NS = 16
NW = NC * NS

EPW = E // NW
B = 112
NFULL = EPW // B
TAIL = EPW - NFULL * B
RPT_M = N // NS

EPT = E // NS
BD = 128
CPT = EPT // BD
BD_TAIL = EPT - CPT * BD
RPT_D = N_PAD // NS

RBLK = 2560


def _sc_mesh():
    return plsc.VectorSubcoreMesh(
        core_axis_name="c", subcore_axis_name="s", num_cores=NC, num_subcores=NS
    )


def _deg_kernel_body(src_hbm, dst_hbm, ones_hbm, zeros_hbm, deg_hbm,
                     idx_v, ones_v, dacc, semA, semB):
    c = lax.axis_index("c")
    s = lax.axis_index("s")
    pltpu.sync_copy(
        zeros_hbm.at[pl.ds(s * RPT_D, RPT_D)], dacc.at[pl.ds(s * RPT_D, RPT_D)]
    )
    pltpu.sync_copy(ones_hbm, ones_v)

    @pl.when(c == 0)
    def _():
        pltpu.sync_copy(src_hbm.at[pl.ds(s * EPT, EPT)], idx_v)

    @pl.when(c == 1)
    def _():
        pltpu.sync_copy(dst_hbm.at[pl.ds(s * EPT, EPT)], idx_v)

    plsc.subcore_barrier()

    def _scat(j, sem):
        return pltpu.async_copy(
            ones_v, dacc.at[idx_v.at[pl.ds(j * BD, BD)]], sem, add=True
        )

    def _scat_wait(j, sem):
        pltpu.make_async_copy(
            ones_v, dacc.at[idx_v.at[pl.ds(j * BD, BD)]], sem
        ).wait()

    _scat(0, semA)

    def body(g, carry):
        j = 2 * g
        _scat(j + 1, semB)
        _scat_wait(j, semA)
        _scat(j + 2, semA)
        _scat_wait(j + 1, semB)
        return carry

    lax.fori_loop(0, CPT // 2 - 1, body, 0)
    _scat_wait(CPT - 2, semA)
    _scat(CPT - 1, semB)
    pltpu.async_copy(
        ones_v.at[pl.ds(0, BD_TAIL)],
        dacc.at[idx_v.at[pl.ds(CPT * BD, BD_TAIL)]],
        semA,
        add=True,
    )
    _scat_wait(CPT - 1, semB)
    pltpu.make_async_copy(
        ones_v.at[pl.ds(0, BD_TAIL)],
        dacc.at[idx_v.at[pl.ds(CPT * BD, BD_TAIL)]],
        semA,
    ).wait()

    plsc.subcore_barrier()
    pltpu.sync_copy(
        dacc.at[pl.ds(s * RPT_D, RPT_D)], deg_hbm.at[c, pl.ds(s * RPT_D, RPT_D)]
    )


def _make_deg_kernel():
    return functools.partial(
        pl.kernel,
        out_type=jax.ShapeDtypeStruct((NC, N_PAD, 8), jnp.float32),
        mesh=_sc_mesh(),
        scratch_types=[
            pltpu.VMEM((EPT,), jnp.int32),
            pltpu.VMEM((BD, 8), jnp.float32),
            pltpu.VMEM_SHARED((N_PAD, 8), jnp.float32),
            pltpu.SemaphoreType.DMA,
            pltpu.SemaphoreType.DMA,
        ],
        compiler_params=pltpu.CompilerParams(use_tc_tiling_on_sc=False),
    )(_deg_kernel_body)


def _msg_kernel_body(
    h_hbm, src_hbm, dst_hbm, zeros_hbm, out_hbm,
    src_v, dst_v, buf0, buf1, acc, sem0, sem1
):
    c = lax.axis_index("c")
    s = lax.axis_index("s")
    w = s * NC + c
    pltpu.sync_copy(src_hbm.at[pl.ds(w * EPW, EPW)], src_v)
    pltpu.sync_copy(dst_hbm.at[pl.ds(w * EPW, EPW)], dst_v)
    pltpu.sync_copy(
        zeros_hbm.at[pl.ds(s * RPT_M, RPT_M)], acc.at[pl.ds(s * RPT_M, RPT_M)]
    )
    plsc.subcore_barrier()

    pltpu.async_copy(h_hbm.at[src_v.at[pl.ds(0, B)]], buf0, sem0)

    def body(g, carry):
        j = 2 * g
        pltpu.async_copy(h_hbm.at[src_v.at[pl.ds((j + 1) * B, B)]], buf1, sem1)
        pltpu.make_async_copy(
            h_hbm.at[src_v.at[pl.ds(j * B, B)]], buf0, sem0
        ).wait()
        pltpu.sync_copy(buf0, acc.at[dst_v.at[pl.ds(j * B, B)]], add=True)
        pltpu.async_copy(h_hbm.at[src_v.at[pl.ds((j + 2) * B, B)]], buf0, sem0)
        pltpu.make_async_copy(
            h_hbm.at[src_v.at[pl.ds((j + 1) * B, B)]], buf1, sem1
        ).wait()
        pltpu.sync_copy(buf1, acc.at[dst_v.at[pl.ds((j + 1) * B, B)]], add=True)
        return carry

    lax.fori_loop(0, (NFULL - 1) // 2, body, 0)
    pltpu.async_copy(
        h_hbm.at[src_v.at[pl.ds(NFULL * B, TAIL)]], buf1.at[pl.ds(0, TAIL)], sem1
    )
    pltpu.make_async_copy(
        h_hbm.at[src_v.at[pl.ds((NFULL - 1) * B, B)]], buf0, sem0
    ).wait()
    pltpu.sync_copy(buf0, acc.at[dst_v.at[pl.ds((NFULL - 1) * B, B)]], add=True)
    pltpu.make_async_copy(
        h_hbm.at[src_v.at[pl.ds(NFULL * B, TAIL)]], buf1.at[pl.ds(0, TAIL)], sem1
    ).wait()
    pltpu.sync_copy(
        buf1.at[pl.ds(0, TAIL)],
        acc.at[dst_v.at[pl.ds(NFULL * B, TAIL)]],
        add=True,
    )
    plsc.subcore_barrier()
    pltpu.sync_copy(
        acc.at[pl.ds(s * RPT_M, RPT_M)], out_hbm.at[c, pl.ds(s * RPT_M, RPT_M)]
    )


def _make_msg_kernel():
    return functools.partial(
        pl.kernel,
        out_type=jax.ShapeDtypeStruct((NC, N, D), jnp.float32),
        mesh=_sc_mesh(),
        scratch_types=[
            pltpu.VMEM((EPW,), jnp.int32),
            pltpu.VMEM((EPW,), jnp.int32),
            pltpu.VMEM((B, D), jnp.float32),
            pltpu.VMEM((B, D), jnp.float32),
            pltpu.VMEM_SHARED((N, D), jnp.float32),
            pltpu.SemaphoreType.DMA,
            pltpu.SemaphoreType.DMA,
        ],
        compiler_params=pltpu.CompilerParams(use_tc_tiling_on_sc=False),
    )(_msg_kernel_body)


def _norms(deg_blk):
    nrm = lax.rsqrt(jnp.maximum(deg_blk[:, :, 0:1], 1.0))
    return nrm[0], nrm[1]


def _tc1_body(x_ref, deg_ref, w_ref, o_ref):
    ns, _ = _norms(deg_ref[...])
    o_ref[...] = jnp.dot(
        x_ref[...] * ns, w_ref[...], preferred_element_type=jnp.float32
    )


def _tc_mid_body(p_ref, deg_ref, b_ref, w_ref, o_ref):
    ns, nd = _norms(deg_ref[...])
    t = (p_ref[0] + p_ref[1]) * nd + b_ref[...]
    t = jnp.where(t > 0.0, t, jnp.exp(jnp.minimum(t, 0.0)) - 1.0)
    o_ref[...] = jnp.dot(t * ns, w_ref[...], preferred_element_type=jnp.float32)


def _tc_out_body(p_ref, deg_ref, b_ref, o_ref):
    _, nd = _norms(deg_ref[...])
    o_ref[...] = (p_ref[0] + p_ref[1]) * nd + b_ref[...]


_GRID = (N_PAD // RBLK,)
_SPEC_ROWS = pl.BlockSpec((RBLK, D), lambda i: (i, 0))
_SPEC_DEG = pl.BlockSpec((NC, RBLK, 8), lambda i: (0, i, 0))
_SPEC_P = pl.BlockSpec((NC, RBLK, D), lambda i: (0, i, 0))
_SPEC_W = pl.BlockSpec((D, D), lambda i: (0, 0))
_SPEC_B = pl.BlockSpec((1, D), lambda i: (0, 0))
_OUT_ROWS = jax.ShapeDtypeStruct((N_PAD, D), jnp.float32)


def kernel(x, edge_index, W1, b1, W2, b2):
    src = edge_index[0]
    dst = edge_index[1]
    zeros_nd = jnp.zeros((N, D), jnp.float32)
    zeros_deg = jnp.zeros((N_PAD, 8), jnp.float32)
    ones_bd = jnp.ones((BD, 8), jnp.float32)

    deg3 = _make_deg_kernel()(src, dst, ones_bd, zeros_deg)

    h1 = pl.pallas_call(
        _tc1_body,
        grid=_GRID,
        in_specs=[_SPEC_ROWS, _SPEC_DEG, _SPEC_W],
        out_specs=_SPEC_ROWS,
        out_shape=_OUT_ROWS,
    )(x, deg3, W1)

    msg = _make_msg_kernel()
    p1 = msg(h1, src, dst, zeros_nd)

    h2 = pl.pallas_call(
        _tc_mid_body,
        grid=_GRID,
        in_specs=[_SPEC_P, _SPEC_DEG, _SPEC_B, _SPEC_W],
        out_specs=_SPEC_ROWS,
        out_shape=_OUT_ROWS,
    )(p1, deg3, b1.reshape(1, D), W2)

    p2 = msg(h2, src, dst, zeros_nd)

    logits = pl.pallas_call(
        _tc_out_body,
        grid=_GRID,
        in_specs=[_SPEC_P, _SPEC_DEG, _SPEC_B],
        out_specs=_SPEC_ROWS,
        out_shape=jax.ShapeDtypeStruct((N, D), jnp.float32),
    )(p2, deg3, b2.reshape(1, D))

    return logits
